# baseline (speedup 1.0000x reference)
import functools

import jax
import jax.numpy as jnp
from jax import lax
from jax.experimental import pallas as pl
from jax.experimental.pallas import tpu as pltpu
from jax.experimental.pallas import tpu_sc as plsc

_NC = 2
_NS = 16
_LANES = 16
_CHUNK = 128


def _sc_gather_body(F, V, spw, inputs_hbm, tables_hbm, w_hbm, x_hbm, wv_hbm,
                    in_v, idx_v, ebuf0, ebuf1, wbuf0, wbuf1,
                    esem0, esem1, wsem0, wsem1, xsem0, xsem1, vsem0, vsem1):
    wid = lax.axis_index("s") * _NC + lax.axis_index("c")
    npairs = spw * F
    p0 = wid * npairs
    nchunk = npairs // _CHUNK

    pltpu.sync_copy(inputs_hbm.at[pl.ds(p0, npairs)], in_v)
    for j in range(nchunk):
        for k in range(_CHUNK // _LANES):
            off = j * _CHUNK + k * _LANES
            pos = lax.iota(jnp.int32, _LANES) + (p0 + off)
            raw = in_v[pl.ds(off, _LANES)]
            idx_v[j, pl.ds(k * _LANES, _LANES)] = raw + lax.rem(pos, F) * V

    ebufs = (ebuf0, ebuf1)
    wbufs = (wbuf0, wbuf1)
    esems = (esem0, esem1)
    wsems = (wsem0, wsem1)
    xsems = (xsem0, xsem1)
    vsems = (vsem0, vsem1)
    edesc = [None, None]
    wdesc = [None, None]
    xdesc = [None, None]
    vdesc = [None, None]

    def fire(j):
        s = j % 2
        ii = idx_v.at[j]
        edesc[s] = pltpu.async_copy(tables_hbm.at[ii], ebufs[s], esems[s])
        wdesc[s] = pltpu.async_copy(w_hbm.at[ii], wbufs[s], wsems[s])

    def writeback(j):
        s = j % 2
        edesc[s].wait()
        wdesc[s].wait()
        row0 = p0 + j * _CHUNK
        xdesc[s] = pltpu.async_copy(ebufs[s],
                                    x_hbm.at[pl.ds(row0, _CHUNK)], xsems[s])
        vdesc[s] = pltpu.async_copy(wbufs[s],
                                    wv_hbm.at[pl.ds(row0, _CHUNK)], vsems[s])

    for j in range(nchunk):
        s = j % 2
        if j >= 2:
            xdesc[s].wait()
            vdesc[s].wait()
        fire(j)
        if j >= 1:
            writeback(j - 1)
    writeback(nchunk - 1)
    xdesc[0].wait()
    vdesc[0].wait()
    xdesc[1].wait()
    vdesc[1].wait()


def _sc_gather(inputs_flat, tables_flat, w_flat, F, V, D):
    BF = inputs_flat.shape[0]
    B = BF // F
    spw = B // (_NC * _NS)
    mesh = plsc.VectorSubcoreMesh(core_axis_name="c", subcore_axis_name="s")
    idx_2d = (spw * F // _CHUNK, _CHUNK)
    kfn = pl.kernel(
        functools.partial(_sc_gather_body, F, V, spw),
        out_type=(
            jax.ShapeDtypeStruct((BF, D), jnp.float32),
            jax.ShapeDtypeStruct((BF,), jnp.float32),
        ),
        mesh=mesh,
        compiler_params=pltpu.CompilerParams(use_tc_tiling_on_sc=False),
        scratch_types=[
            pltpu.VMEM((spw * F,), jnp.int32),
            pltpu.VMEM(idx_2d, jnp.int32),
            pltpu.VMEM((_CHUNK, D), jnp.float32),
            pltpu.VMEM((_CHUNK, D), jnp.float32),
            pltpu.VMEM((_CHUNK,), jnp.float32),
            pltpu.VMEM((_CHUNK,), jnp.float32),
        ] + [pltpu.SemaphoreType.DMA] * 8,
    )
    return kfn(inputs_flat, tables_flat, w_flat)


def _tc_dnn_body(x_ref, wv_ref, W1_ref, b1_ref, W2_ref, b2_ref,
                 W3_ref, b3_ref, Wf_ref, bf_ref, o_ref):
    xb = (x_ref[...] * (2.0 ** 8)).astype(jnp.float8_e4m3fn)
    acc = jnp.dot(xb, W1_ref[...],
                  preferred_element_type=jnp.float32) * (2.0 ** -13)
    h = jnp.maximum(acc + b1_ref[...], 0.0).astype(jnp.bfloat16)
    h = jnp.maximum(
        jnp.dot(h, W2_ref[...], preferred_element_type=jnp.float32)
        + b2_ref[...], 0.0).astype(jnp.bfloat16)
    h = jnp.maximum(
        jnp.dot(h, W3_ref[...], preferred_element_type=jnp.float32)
        + b3_ref[...], 0.0)
    d = jnp.sum(h * Wf_ref[...], axis=1, keepdims=True) + bf_ref[0, 0]
    wide = jnp.sum(wv_ref[...], axis=1, keepdims=True)
    o_ref[...] = jax.nn.sigmoid(0.5 * wide + 0.5 * d)


def _tc_dnn(x, wv, W1, b1, W2, b2, W3, b3, Wf, bf):
    B, DIN = x.shape
    F = wv.shape[1]
    H1, H2, H3 = W1.shape[1], W2.shape[1], W3.shape[1]
    BM = 1024
    grid = (B // BM,)
    return pl.pallas_call(
        _tc_dnn_body,
        grid=grid,
        in_specs=[
            pl.BlockSpec((BM, DIN), lambda i: (i, 0)),
            pl.BlockSpec((BM, F), lambda i: (i, 0)),
            pl.BlockSpec((DIN, H1), lambda i: (0, 0)),
            pl.BlockSpec((1, H1), lambda i: (0, 0)),
            pl.BlockSpec((H1, H2), lambda i: (0, 0)),
            pl.BlockSpec((1, H2), lambda i: (0, 0)),
            pl.BlockSpec((H2, H3), lambda i: (0, 0)),
            pl.BlockSpec((1, H3), lambda i: (0, 0)),
            pl.BlockSpec((1, H3), lambda i: (0, 0)),
            pl.BlockSpec((1, 1), lambda i: (0, 0)),
        ],
        out_specs=pl.BlockSpec((BM, 1), lambda i: (i, 0)),
        out_shape=jax.ShapeDtypeStruct((B, 1), jnp.float32),
    )(x, wv, W1, b1, W2, b2, W3, b3, Wf, bf)


def kernel(inputs, embed_tables, w_lin, W1, b1, W2, b2, W3, b3, Wf, bf):
    B, F = inputs.shape
    _, V, D = embed_tables.shape
    tables_flat = embed_tables.reshape(F * V, D)
    inputs_flat = inputs.reshape(B * F)

    H1 = W1.shape[1]
    W1b = (W1 * (2.0 ** 5)).astype(jnp.float8_e4m3fn)
    W2b = W2.astype(jnp.bfloat16)
    W3b = W3.astype(jnp.bfloat16)
    w_flat = w_lin.reshape(F * V)

    nsplit = 1
    Bs = B // nsplit
    outs = []
    gathered = [
        _sc_gather(inputs_flat[i * Bs * F:(i + 1) * Bs * F], tables_flat,
                   w_flat, F, V, D)
        for i in range(nsplit)
    ]
    for x_rows, wv in gathered:
        outs.append(_tc_dnn(x_rows.reshape(Bs, F * D), wv.reshape(Bs, F),
                            W1b, b1.reshape(1, H1),
                            W2b, b2.reshape(1, -1),
                            W3b, b3.reshape(1, -1),
                            Wf.reshape(1, -1), bf.reshape(1, 1)))
    return jnp.concatenate(outs, axis=0)


# speedup vs baseline: 1.3296x; 1.3296x over previous
import functools

import jax
import jax.numpy as jnp
from jax import lax
from jax.experimental import pallas as pl
from jax.experimental.pallas import tpu as pltpu
from jax.experimental.pallas import tpu_sc as plsc

_NC = 2
_NS = 16
_LANES = 16


def _sc_gather_body(F, V, B, spw, inT_hbm, tables_hbm, w_hbm, x26_hbm,
                    wv_hbm, in_v, idx_v, ebuf0, ebuf1, wbuf0, wbuf1,
                    insem, esem0, esem1, wsem0, wsem1,
                    xsem0, xsem1, vsem0, vsem1):
    wid = lax.axis_index("s") * _NC + lax.axis_index("c")
    s0 = wid * spw

    indesc = [
        pltpu.async_copy(inT_hbm.at[pl.ds(f * B + s0, spw)],
                         in_v.at[pl.ds(f * spw, spw)], insem)
        for f in range(F)
    ]
    for d in indesc:
        d.wait()
    for f in range(F):
        for k in range(spw // _LANES):
            off = f * spw + k * _LANES
            idx_v[f, pl.ds(k * _LANES, _LANES)] = \
                in_v[pl.ds(off, _LANES)] + (f * V)

    ebufs = (ebuf0, ebuf1)
    wbufs = (wbuf0, wbuf1)
    esems = (esem0, esem1)
    wsems = (wsem0, wsem1)
    xsems = (xsem0, xsem1)
    vsems = (vsem0, vsem1)
    edesc = [None, None]
    wdesc = [None, None]
    xdesc = [None, None]
    vdesc = [None, None]

    def fire(f):
        s = f % 2
        ii = idx_v.at[f]
        edesc[s] = pltpu.async_copy(tables_hbm.at[ii], ebufs[s], esems[s])
        wdesc[s] = pltpu.async_copy(w_hbm.at[ii], wbufs[s], wsems[s])

    def writeback(f):
        s = f % 2
        edesc[s].wait()
        wdesc[s].wait()
        xdesc[s] = pltpu.async_copy(
            ebufs[s], x26_hbm.at[f, pl.ds(s0, spw)], xsems[s])
        vdesc[s] = pltpu.async_copy(
            wbufs[s], wv_hbm.at[pl.ds(f * B + s0, spw)], vsems[s])

    for f in range(F):
        s = f % 2
        if f >= 2:
            xdesc[s].wait()
            vdesc[s].wait()
        fire(f)
        if f >= 1:
            writeback(f - 1)
    writeback(F - 1)
    xdesc[0].wait()
    vdesc[0].wait()
    xdesc[1].wait()
    vdesc[1].wait()


def _sc_gather(inputs_T_flat, tables_flat, w_flat, F, V, D):
    BF = inputs_T_flat.shape[0]
    B = BF // F
    spw = B // (_NC * _NS)
    mesh = plsc.VectorSubcoreMesh(core_axis_name="c", subcore_axis_name="s")
    kfn = pl.kernel(
        functools.partial(_sc_gather_body, F, V, B, spw),
        out_type=(
            jax.ShapeDtypeStruct((F, B, D), jnp.float32),
            jax.ShapeDtypeStruct((BF,), jnp.float32),
        ),
        mesh=mesh,
        compiler_params=pltpu.CompilerParams(use_tc_tiling_on_sc=False),
        scratch_types=[
            pltpu.VMEM((spw * F,), jnp.int32),
            pltpu.VMEM((F, spw), jnp.int32),
            pltpu.VMEM((spw, D), jnp.float32),
            pltpu.VMEM((spw, D), jnp.float32),
            pltpu.VMEM((spw,), jnp.float32),
            pltpu.VMEM((spw,), jnp.float32),
        ] + [pltpu.SemaphoreType.DMA] * 9,
    )
    return kfn(inputs_T_flat, tables_flat, w_flat)


def _tc_dnn_body(F, x_ref, wv_ref, W1_ref, b1_ref, W2_ref, b2_ref,
                 W3_ref, b3_ref, Wf_ref, bf_ref, o_ref):
    bm = x_ref.shape[1]
    h1 = W1_ref.shape[1]
    acc = jnp.zeros((bm, h1), jnp.float32)
    for t in range(F // 2):
        xp = jnp.concatenate([x_ref[2 * t], x_ref[2 * t + 1]], axis=1)
        xp8 = (xp * (2.0 ** 8)).astype(jnp.float8_e4m3fn)
        wp = W1_ref[pl.ds(t * 256, 256), :]
        acc = acc + jnp.dot(xp8, wp, preferred_element_type=jnp.float32)
    h = jnp.maximum(acc * (2.0 ** -13) + b1_ref[...], 0.0).astype(jnp.bfloat16)
    h = jnp.maximum(
        jnp.dot(h, W2_ref[...], preferred_element_type=jnp.float32)
        + b2_ref[...], 0.0).astype(jnp.bfloat16)
    h = jnp.maximum(
        jnp.dot(h, W3_ref[...], preferred_element_type=jnp.float32)
        + b3_ref[...], 0.0)
    d = jnp.sum(h * Wf_ref[...], axis=1, keepdims=True) + bf_ref[0, 0]
    wide = jnp.sum(wv_ref[...], axis=0)[:, None]
    o_ref[...] = jax.nn.sigmoid(0.5 * wide + 0.5 * d)


def _tc_dnn(x26, wv2, W1, b1, W2, b2, W3, b3, Wf, bf):
    F, B, D = x26.shape
    DIN = W1.shape[0]
    H1, H2, H3 = W1.shape[1], W2.shape[1], W3.shape[1]
    BM = 1024
    grid = (B // BM,)
    return pl.pallas_call(
        functools.partial(_tc_dnn_body, F),
        grid=grid,
        in_specs=[
            pl.BlockSpec((F, BM, D), lambda i: (0, i, 0)),
            pl.BlockSpec((F, BM), lambda i: (0, i)),
            pl.BlockSpec((DIN, H1), lambda i: (0, 0)),
            pl.BlockSpec((1, H1), lambda i: (0, 0)),
            pl.BlockSpec((H1, H2), lambda i: (0, 0)),
            pl.BlockSpec((1, H2), lambda i: (0, 0)),
            pl.BlockSpec((H2, H3), lambda i: (0, 0)),
            pl.BlockSpec((1, H3), lambda i: (0, 0)),
            pl.BlockSpec((1, H3), lambda i: (0, 0)),
            pl.BlockSpec((1, 1), lambda i: (0, 0)),
        ],
        out_specs=pl.BlockSpec((BM, 1), lambda i: (i, 0)),
        out_shape=jax.ShapeDtypeStruct((B, 1), jnp.float32),
    )(x26, wv2, W1, b1, W2, b2, W3, b3, Wf, bf)


def kernel(inputs, embed_tables, w_lin, W1, b1, W2, b2, W3, b3, Wf, bf):
    B, F = inputs.shape
    _, V, D = embed_tables.shape
    tables_flat = embed_tables.reshape(F * V, D)
    inputs_T_flat = inputs.T.reshape(B * F)

    x26, wv = _sc_gather(inputs_T_flat, tables_flat, w_lin.reshape(F * V),
                         F, V, D)
    wv2 = wv.reshape(F, B)

    H1 = W1.shape[1]
    out = _tc_dnn(x26, wv2,
                  (W1 * (2.0 ** 5)).astype(jnp.float8_e4m3fn),
                  b1.reshape(1, H1),
                  W2.astype(jnp.bfloat16), b2.reshape(1, -1),
                  W3.astype(jnp.bfloat16), b3.reshape(1, -1),
                  Wf.reshape(1, -1), bf.reshape(1, 1))
    return out


# speedup vs baseline: 1.3714x; 1.0315x over previous
import functools

import jax
import jax.numpy as jnp
from jax import lax
from jax.experimental import pallas as pl
from jax.experimental.pallas import tpu as pltpu
from jax.experimental.pallas import tpu_sc as plsc

_NC = 2
_NS = 16
_LANES = 16


def _sc_gather_body(F, V, B, spw, inT_hbm, tables_hbm, w_hbm, x26_hbm,
                    wv_hbm, in_v, idx_v, wv_all,
                    ebuf0, ebuf1, ebuf2, ebuf3,
                    esem0, esem1, esem2, esem3,
                    xsem0, xsem1, xsem2, xsem3, wsem, vsem):
    wid = lax.axis_index("s") * _NC + lax.axis_index("c")
    s0 = wid * spw
    npairs = spw * F

    pltpu.sync_copy(inT_hbm.at[pl.ds(wid * npairs, npairs)], in_v)
    for f in range(F):
        for k in range(spw // _LANES):
            off = f * spw + k * _LANES
            idx_v[f, pl.ds(k * _LANES, _LANES)] = \
                in_v[pl.ds(off, _LANES)] + (f * V)

    ebufs = (ebuf0, ebuf1, ebuf2, ebuf3)
    esems = (esem0, esem1, esem2, esem3)
    xsems = (xsem0, xsem1, xsem2, xsem3)
    nslot = 4
    edesc = [None] * nslot
    xdesc = [None] * nslot
    wdesc = [None] * F
    vdesc = [None] * F

    for f in range(F):
        s = f % nslot
        if f >= nslot:
            xdesc[s].wait()
        edesc[s] = pltpu.async_copy(tables_hbm.at[idx_v.at[f]],
                                    ebufs[s], esems[s])
        if f >= 8:
            wdesc[f - 8].wait()
        wdesc[f] = pltpu.async_copy(
            w_hbm.at[idx_v.at[f]], wv_all.at[pl.ds(f * spw, spw)], wsem)
        if f >= 1:
            p = (f - 1) % nslot
            edesc[p].wait()
            xdesc[p] = pltpu.async_copy(
                ebufs[p], x26_hbm.at[f - 1, pl.ds(s0, spw)], xsems[p])
    p = (F - 1) % nslot
    edesc[p].wait()
    xdesc[p] = pltpu.async_copy(
        ebufs[p], x26_hbm.at[F - 1, pl.ds(s0, spw)], xsems[p])
    for f in range(F - 8, F):
        wdesc[f].wait()
    for f in range(F):
        if f >= 8:
            vdesc[f - 8].wait()
        vdesc[f] = pltpu.async_copy(
            wv_all.at[pl.ds(f * spw, spw)],
            wv_hbm.at[pl.ds(f * B + s0, spw)], vsem)
    for f in range(F - 8, F):
        vdesc[f].wait()
    for s in range(nslot):
        xdesc[s].wait()


def _sc_gather(inputs_T_flat, tables_flat, w_flat, F, V, D):
    BF = inputs_T_flat.shape[0]
    B = BF // F
    spw = B // (_NC * _NS)
    mesh = plsc.VectorSubcoreMesh(core_axis_name="c", subcore_axis_name="s")
    kfn = pl.kernel(
        functools.partial(_sc_gather_body, F, V, B, spw),
        out_type=(
            jax.ShapeDtypeStruct((F, B, D), jnp.float32),
            jax.ShapeDtypeStruct((BF,), jnp.float32),
        ),
        mesh=mesh,
        compiler_params=pltpu.CompilerParams(use_tc_tiling_on_sc=False),
        scratch_types=[
            pltpu.VMEM((spw * F,), jnp.int32),
            pltpu.VMEM((F, spw), jnp.int32),
            pltpu.VMEM((spw * F,), jnp.float32),
            pltpu.VMEM((spw, D), jnp.float32),
            pltpu.VMEM((spw, D), jnp.float32),
            pltpu.VMEM((spw, D), jnp.float32),
            pltpu.VMEM((spw, D), jnp.float32),
        ] + [pltpu.SemaphoreType.DMA] * 10,
    )
    return kfn(inputs_T_flat, tables_flat, w_flat)


def _tc_dnn_body(F, x_ref, wv_ref, W1_ref, b1_ref, W2_ref, b2_ref,
                 W3_ref, b3_ref, Wf_ref, bf_ref, o_ref):
    bm = x_ref.shape[1]
    h1 = W1_ref.shape[1]
    acc = jnp.zeros((bm, h1), jnp.float32)
    for t in range(F // 2):
        xp = jnp.concatenate([x_ref[2 * t], x_ref[2 * t + 1]], axis=1)
        xp8 = (xp * (2.0 ** 8)).astype(jnp.float8_e4m3fn)
        wp = W1_ref[pl.ds(t * 256, 256), :]
        acc = acc + jnp.dot(xp8, wp, preferred_element_type=jnp.float32)
    h = jnp.maximum(acc * (2.0 ** -13) + b1_ref[...], 0.0).astype(jnp.bfloat16)
    h = jnp.maximum(
        jnp.dot(h, W2_ref[...], preferred_element_type=jnp.float32)
        + b2_ref[...], 0.0).astype(jnp.bfloat16)
    h = jnp.maximum(
        jnp.dot(h, W3_ref[...], preferred_element_type=jnp.float32)
        + b3_ref[...], 0.0)
    d = jnp.sum(h * Wf_ref[...], axis=1, keepdims=True) + bf_ref[0, 0]
    wide = jnp.sum(wv_ref[...], axis=0)[:, None]
    o_ref[...] = jax.nn.sigmoid(0.5 * wide + 0.5 * d)


def _tc_dnn(x26, wv2, W1, b1, W2, b2, W3, b3, Wf, bf):
    F, B, D = x26.shape
    DIN = W1.shape[0]
    H1, H2, H3 = W1.shape[1], W2.shape[1], W3.shape[1]
    BM = 1024
    grid = (B // BM,)
    return pl.pallas_call(
        functools.partial(_tc_dnn_body, F),
        grid=grid,
        in_specs=[
            pl.BlockSpec((F, BM, D), lambda i: (0, i, 0)),
            pl.BlockSpec((F, BM), lambda i: (0, i)),
            pl.BlockSpec((DIN, H1), lambda i: (0, 0)),
            pl.BlockSpec((1, H1), lambda i: (0, 0)),
            pl.BlockSpec((H1, H2), lambda i: (0, 0)),
            pl.BlockSpec((1, H2), lambda i: (0, 0)),
            pl.BlockSpec((H2, H3), lambda i: (0, 0)),
            pl.BlockSpec((1, H3), lambda i: (0, 0)),
            pl.BlockSpec((1, H3), lambda i: (0, 0)),
            pl.BlockSpec((1, 1), lambda i: (0, 0)),
        ],
        out_specs=pl.BlockSpec((BM, 1), lambda i: (i, 0)),
        out_shape=jax.ShapeDtypeStruct((B, 1), jnp.float32),
    )(x26, wv2, W1, b1, W2, b2, W3, b3, Wf, bf)


def kernel(inputs, embed_tables, w_lin, W1, b1, W2, b2, W3, b3, Wf, bf):
    B, F = inputs.shape
    _, V, D = embed_tables.shape
    tables_flat = embed_tables.reshape(F * V, D)
    nw = _NC * _NS
    spw = B // nw
    inputs_T_flat = inputs.reshape(nw, spw, F).transpose(0, 2, 1).reshape(B * F)

    x26, wv = _sc_gather(inputs_T_flat, tables_flat, w_lin.reshape(F * V),
                         F, V, D)
    wv2 = wv.reshape(F, B)

    H1 = W1.shape[1]
    out = _tc_dnn(x26, wv2,
                  (W1 * (2.0 ** 5)).astype(jnp.float8_e4m3fn),
                  b1.reshape(1, H1),
                  W2.astype(jnp.bfloat16), b2.reshape(1, -1),
                  W3.astype(jnp.bfloat16), b3.reshape(1, -1),
                  Wf.reshape(1, -1), bf.reshape(1, 1))
    return out
